# baseline (speedup 1.0000x reference)
import math

import jax
import jax.numpy as jnp
from jax.experimental import pallas as pl

_EPS = 1e-9
_K = 10
_HI = jax.lax.Precision.HIGHEST


def _encoder_body(scores_ref, dec_t_ref, anc_t_ref, gtb_ref, gtb_aug_ref,
                  lab_col_ref, mask_col_ref, at_dec_ref, at_gt_ref,
                  bbox_ref, cls_ref, match_ref):
    G = gtb_ref.shape[0]
    A, C = scores_ref.shape

    dec = dec_t_ref[...]
    anc = anc_t_ref[...]
    gtb = gtb_ref[...]
    lab_col = lab_col_ref[...]
    mask_col = mask_col_ref[...]

    oh_gc = (lab_col == jax.lax.broadcasted_iota(jnp.int32, (G, C), 1)).astype(jnp.float32)
    bscores = jax.lax.dot_general(oh_gc, scores_ref[...], (((1,), (1,)), ((), ())),
                                  precision=_HI, preferred_element_type=jnp.float32)

    xmin1, ymin1, xmax1, ymax1 = (gtb[:, i:i + 1] for i in range(4))
    xmin2, ymin2, xmax2, ymax2 = (dec[i:i + 1, :] for i in range(4))
    w1 = xmax1 - xmin1
    h1 = ymax1 - ymin1 + _EPS
    w2 = xmax2 - xmin2
    h2 = ymax2 - ymin2 + _EPS
    inter = (jnp.maximum(jnp.minimum(xmax1, xmax2) - jnp.maximum(xmin1, xmin2), 0.0)
             * jnp.maximum(jnp.minimum(ymax1, ymax2) - jnp.maximum(ymin1, ymin2), 0.0))
    union = w1 * h1 + w2 * h2 - inter + _EPS
    iou = inter / union
    convex = ((jnp.maximum(xmax1, xmax2) - jnp.minimum(xmin1, xmin2)) ** 2
              + (jnp.maximum(ymax1, ymax2) - jnp.minimum(ymin1, ymin2)) ** 2)
    cdist = (((xmin1 + xmax1) / 2 - (xmin2 + xmax2) / 2) ** 2
             + ((ymin1 + ymax1) / 2 - (ymin2 + ymax2) / 2) ** 2)
    v = (4.0 / (math.pi ** 2)) * (at_dec_ref[...] - at_gt_ref[...]) ** 2
    alpha = v / (v - iou + (1.0 + _EPS))
    ciou = iou - (cdist / (convex + _EPS) + v * alpha)

    ax = anc[0:1, :]
    ay = anc[1:2, :]
    matching = (xmin1 < ax) & (ymin1 < ay) & (xmax1 > ax) & (ymax1 > ay)
    valid = matching & (mask_col > 0.0)
    ov3 = (ciou * ciou) * ciou
    ov6 = ov3 * ov3
    metrics = jnp.where(valid, jnp.sqrt(bscores) * ov6, 0.0)

    work = metrics
    thr = None
    for _ in range(_K):
        thr = jnp.max(work, axis=1, keepdims=True)
        work = jnp.where(work >= thr, -1.0, work)
    sel = (metrics >= thr) & (metrics > 0.0)

    m_ov = jnp.where(sel, ciou, 0.0)
    m_met = jnp.where(sel, metrics, 0.0)

    max_align = jnp.max(m_met, axis=1, keepdims=True)
    max_ov_g = jnp.max(m_ov, axis=1, keepdims=True)
    norm_align = jnp.max(m_met * (max_ov_g / (max_align + _EPS)),
                         axis=0, keepdims=True)

    maxov_a = jnp.max(m_ov, axis=0, keepdims=True)
    giota = jax.lax.broadcasted_iota(jnp.int32, (G, A), 0)
    gt_match = jnp.min(jnp.where(m_ov == maxov_a, giota, G), axis=0, keepdims=True)
    ok = maxov_a > 0.0

    sel_row = jnp.where(ok, gt_match, G)
    oh_aug = (jax.lax.broadcasted_iota(jnp.int32, (G + 1, A), 0) == sel_row
              ).astype(jnp.float32)
    bbox_ref[...] = jax.lax.dot_general(
        oh_aug, gtb_aug_ref[...], (((0,), (0,)), ((), ())),
        precision=_HI, preferred_element_type=jnp.float32)

    scaled = jnp.where((giota == gt_match) & ok,
                       jnp.broadcast_to(norm_align, (G, A)), 0.0)
    cls_ref[...] = jax.lax.dot_general(
        scaled, oh_gc, (((0,), (0,)), ((), ())),
        precision=_HI, preferred_element_type=jnp.float32)

    match_ref[...] = (gt_match > 0).astype(jnp.float32)


def _encode(scores, decode_bboxes, anchors, ground_truth_labels,
            ground_truth_bboxes, ground_truth_mask, interpret=False):
    B, A, C = scores.shape
    G = ground_truth_labels.shape[1]

    dec_t = jnp.transpose(decode_bboxes, (0, 2, 1))
    anc_t = jnp.transpose(anchors, (1, 0))
    gtb = ground_truth_bboxes
    gtb_aug = jnp.concatenate(
        [gtb, jnp.full((B, 1, 4), -1.0, dtype=gtb.dtype)], axis=1)
    lab_col = ground_truth_labels.astype(jnp.int32)[:, :, None]
    mask_col = ground_truth_mask.astype(jnp.float32)
    at_dec = jnp.arctan((decode_bboxes[..., 2] - decode_bboxes[..., 0])
                        / (decode_bboxes[..., 3] - decode_bboxes[..., 1] + _EPS))[:, None, :]
    at_gt = jnp.arctan((gtb[..., 2] - gtb[..., 0])
                       / (gtb[..., 3] - gtb[..., 1] + _EPS))[:, :, None]

    out_shape = (
        jax.ShapeDtypeStruct((B, A, 4), jnp.float32),
        jax.ShapeDtypeStruct((B, A, C), jnp.float32),
        jax.ShapeDtypeStruct((B, 1, A), jnp.float32),
    )
    in_specs = [
        pl.BlockSpec((None, A, C), lambda b: (b, 0, 0)),
        pl.BlockSpec((None, 4, A), lambda b: (b, 0, 0)),
        pl.BlockSpec((2, A), lambda b: (0, 0)),
        pl.BlockSpec((None, G, 4), lambda b: (b, 0, 0)),
        pl.BlockSpec((None, G + 1, 4), lambda b: (b, 0, 0)),
        pl.BlockSpec((None, G, 1), lambda b: (b, 0, 0)),
        pl.BlockSpec((None, G, 1), lambda b: (b, 0, 0)),
        pl.BlockSpec((None, 1, A), lambda b: (b, 0, 0)),
        pl.BlockSpec((None, G, 1), lambda b: (b, 0, 0)),
    ]
    out_specs = (
        pl.BlockSpec((None, A, 4), lambda b: (b, 0, 0)),
        pl.BlockSpec((None, A, C), lambda b: (b, 0, 0)),
        pl.BlockSpec((None, 1, A), lambda b: (b, 0, 0)),
    )
    bbox_labels, class_labels, match = pl.pallas_call(
        _encoder_body,
        grid=(B,),
        in_specs=in_specs,
        out_specs=out_specs,
        out_shape=out_shape,
        interpret=interpret,
    )(scores, dec_t, anc_t, gtb, gtb_aug, lab_col, mask_col, at_dec, at_gt)

    return bbox_labels, class_labels, match[:, 0, :]


def kernel(scores, decode_bboxes, anchors, ground_truth_labels,
           ground_truth_bboxes, ground_truth_mask):
    return _encode(scores, decode_bboxes, anchors, ground_truth_labels,
                   ground_truth_bboxes, ground_truth_mask)


# speedup vs baseline: 1.7104x; 1.7104x over previous
import math

import jax
import jax.numpy as jnp
from jax.experimental import pallas as pl
from jax.experimental.pallas import tpu as pltpu

_EPS = 1e-9
_K = 10


_TA = 512


def _encoder_body(scores_t_ref, dec_t_ref, anc_t_ref, gtb_ref, gtb_t_ref,
                  lab_row_ref, mask_col_ref, at_dec_ref, at_gt_ref,
                  lab_smem_ref, bbox_ref, cls_ref, match_ref,
                  bscores_ref, ciou_ref, met_ref):
    G = gtb_ref.shape[0]
    C, A = scores_t_ref.shape

    def _gather_row(g, carry):
        lab = lab_smem_ref[0, g]
        bscores_ref[pl.ds(g, 1), :] = scores_t_ref[pl.ds(lab, 1), :]
        return carry
    jax.lax.fori_loop(0, G, _gather_row, 0, unroll=4)

    n_full_a = A // _TA
    a_tiles = [(i * _TA, _TA) for i in range(n_full_a)]
    if A % _TA:
        a_tiles.append((n_full_a * _TA, A % _TA))

    def _pair_block(g0, gh):
        gtb_blk = gtb_ref[pl.ds(g0, gh), :]
        xmin1 = gtb_blk[:, 0:1]
        ymin1 = gtb_blk[:, 1:2]
        xmax1 = gtb_blk[:, 2:3]
        ymax1 = gtb_blk[:, 3:4]
        w1 = xmax1 - xmin1
        h1 = ymax1 - ymin1 + _EPS
        w1h1 = w1 * h1
        cx1 = (xmin1 + xmax1) / 2
        cy1 = (ymin1 + ymax1) / 2
        at1 = at_gt_ref[pl.ds(g0, gh), :]
        mvalid = mask_col_ref[pl.ds(g0, gh), :] > 0.0
        for a0, ta in a_tiles:
            asl = pl.ds(a0, ta)
            xmin2 = dec_t_ref[0:1, asl]
            ymin2 = dec_t_ref[1:2, asl]
            xmax2 = dec_t_ref[2:3, asl]
            ymax2 = dec_t_ref[3:4, asl]
            w2 = xmax2 - xmin2
            h2 = ymax2 - ymin2 + _EPS
            inter = (jnp.maximum(jnp.minimum(xmax1, xmax2) - jnp.maximum(xmin1, xmin2), 0.0)
                     * jnp.maximum(jnp.minimum(ymax1, ymax2) - jnp.maximum(ymin1, ymin2), 0.0))
            union = w1h1 + w2 * h2 - inter + _EPS
            iou = inter / union
            convex = ((jnp.maximum(xmax1, xmax2) - jnp.minimum(xmin1, xmin2)) ** 2
                      + (jnp.maximum(ymax1, ymax2) - jnp.minimum(ymin1, ymin2)) ** 2)
            cdist = ((cx1 - (xmin2 + xmax2) / 2) ** 2
                     + (cy1 - (ymin2 + ymax2) / 2) ** 2)
            v = (4.0 / (math.pi ** 2)) * (at_dec_ref[0:1, asl] - at1) ** 2
            alpha = v / (v - iou + (1.0 + _EPS))
            ciou_t = iou - (cdist / (convex + _EPS) + v * alpha)

            ax = anc_t_ref[0:1, asl]
            ay = anc_t_ref[1:2, asl]
            valid = ((xmin1 < ax) & (ymin1 < ay) & (xmax1 > ax) & (ymax1 > ay)
                     & mvalid)
            ov3 = (ciou_t * ciou_t) * ciou_t
            ov6 = ov3 * ov3
            met_t = jnp.where(
                valid, jnp.sqrt(bscores_ref[pl.ds(g0, gh), asl]) * ov6, 0.0)
            ciou_ref[pl.ds(g0, gh), asl] = ciou_t
            met_ref[pl.ds(g0, gh), asl] = met_t

    def _pair_loop(gi, carry):
        _pair_block(pl.multiple_of(gi * 8, 8), 8)
        return carry
    jax.lax.fori_loop(0, G // 8, _pair_loop, 0)
    if G % 8:
        _pair_block((G // 8) * 8, G % 8)

    ciou = ciou_ref[...]
    metrics = met_ref[...]

    work = metrics
    thr = None
    for _ in range(_K):
        thr = jnp.max(work, axis=1, keepdims=True)
        work = jnp.where(work >= thr, -1.0, work)
    sel = (metrics >= thr) & (metrics > 0.0)

    m_ov = jnp.where(sel, ciou, 0.0)
    m_met = jnp.where(sel, metrics, 0.0)

    max_align = jnp.max(m_met, axis=1, keepdims=True)
    max_ov_g = jnp.max(m_ov, axis=1, keepdims=True)
    norm_align = jnp.max(m_met * (max_ov_g / (max_align + _EPS)),
                         axis=0, keepdims=True)

    maxov_a = jnp.max(m_ov, axis=0, keepdims=True)
    giota = jax.lax.broadcasted_iota(jnp.int32, (G, A), 0)
    gt_match = jnp.min(jnp.where(m_ov == maxov_a, giota, G), axis=0, keepdims=True)
    ok = maxov_a > 0.0

    oh_ga = (giota == gt_match).astype(jnp.float32)
    bbox = jax.lax.dot(gtb_t_ref[...], oh_ga, precision=jax.lax.Precision.HIGHEST,
                       preferred_element_type=jnp.float32)
    bbox_ref[...] = jnp.where(ok, bbox, -1.0)

    oh_cg = (lab_row_ref[...] == jax.lax.broadcasted_iota(jnp.int32, (C, G), 0)).astype(jnp.float32)
    cls = jax.lax.dot(oh_cg, oh_ga, preferred_element_type=jnp.float32)
    cls_ref[...] = cls * jnp.where(ok, norm_align, 0.0)
    match_ref[...] = (gt_match > 0).astype(jnp.float32)


def _encode(scores, decode_bboxes, anchors, ground_truth_labels,
            ground_truth_bboxes, ground_truth_mask, interpret=False):
    B, A, C = scores.shape
    G = ground_truth_labels.shape[1]

    scores_t = jnp.transpose(scores, (0, 2, 1))
    dec_t = jnp.transpose(decode_bboxes, (0, 2, 1))
    anc_t = jnp.transpose(anchors, (1, 0))
    gtb = ground_truth_bboxes
    gtb_t = jnp.transpose(gtb, (0, 2, 1))
    lab_row = ground_truth_labels[:, None, :].astype(jnp.int32)
    lab_flat = ground_truth_labels.astype(jnp.int32)[:, None, :]
    mask_col = ground_truth_mask.astype(jnp.float32)
    at_dec = jnp.arctan((decode_bboxes[..., 2] - decode_bboxes[..., 0])
                        / (decode_bboxes[..., 3] - decode_bboxes[..., 1] + _EPS))[:, None, :]
    at_gt = jnp.arctan((gtb[..., 2] - gtb[..., 0])
                       / (gtb[..., 3] - gtb[..., 1] + _EPS))[:, :, None]

    out_shape = (
        jax.ShapeDtypeStruct((B, 4, A), jnp.float32),
        jax.ShapeDtypeStruct((B, C, A), jnp.float32),
        jax.ShapeDtypeStruct((B, 1, A), jnp.float32),
    )
    in_specs = [
        pl.BlockSpec((None, C, A), lambda b: (b, 0, 0)),
        pl.BlockSpec((None, 4, A), lambda b: (b, 0, 0)),
        pl.BlockSpec((2, A), lambda b: (0, 0)),
        pl.BlockSpec((None, G, 4), lambda b: (b, 0, 0)),
        pl.BlockSpec((None, 4, G), lambda b: (b, 0, 0)),
        pl.BlockSpec((None, 1, G), lambda b: (b, 0, 0)),
        pl.BlockSpec((None, G, 1), lambda b: (b, 0, 0)),
        pl.BlockSpec((None, 1, A), lambda b: (b, 0, 0)),
        pl.BlockSpec((None, G, 1), lambda b: (b, 0, 0)),
        pl.BlockSpec((None, 1, G), lambda b: (b, 0, 0), memory_space=pltpu.SMEM),
    ]
    out_specs = (
        pl.BlockSpec((None, 4, A), lambda b: (b, 0, 0)),
        pl.BlockSpec((None, C, A), lambda b: (b, 0, 0)),
        pl.BlockSpec((None, 1, A), lambda b: (b, 0, 0)),
    )
    bbox_t, cls_t, match = pl.pallas_call(
        _encoder_body,
        grid=(B,),
        in_specs=in_specs,
        out_specs=out_specs,
        out_shape=out_shape,
        scratch_shapes=[pltpu.VMEM((G, A), jnp.float32),
                        pltpu.VMEM((G, A), jnp.float32),
                        pltpu.VMEM((G, A), jnp.float32)],
        interpret=interpret,
    )(scores_t, dec_t, anc_t, gtb, gtb_t, lab_row, mask_col, at_dec, at_gt, lab_flat)

    bbox_labels = jnp.transpose(bbox_t, (0, 2, 1))
    class_labels = jnp.transpose(cls_t, (0, 2, 1))
    return bbox_labels, class_labels, match[:, 0, :]


def kernel(scores, decode_bboxes, anchors, ground_truth_labels,
           ground_truth_bboxes, ground_truth_mask):
    return _encode(scores, decode_bboxes, anchors, ground_truth_labels,
                   ground_truth_bboxes, ground_truth_mask)
